# baseline (speedup 1.0000x reference)
import functools

import jax
import jax.numpy as jnp
from jax.experimental import pallas as pl
from jax.experimental.pallas import tpu as pltpu

_M_RANGE = 3
_K = 7
_NC = _K * _K


def _body(gt_ref, im_ref, kt_ref, kt3_ref, mm_ref, pred_ref, mxs_ref,
          *, bh, h, w):
    rows = bh + 2 * _M_RANGE
    wp = w + 2 * _M_RANGE

    gt = gt_ref[0, 0]
    mx_ = gt[0]
    my_ = gt[1]
    fy = jnp.floor(my_)
    gy = my_ - fy
    iy = fy.astype(jnp.int32) + _M_RANGE
    fx = jnp.floor(mx_)
    gx = mx_ - fx
    ix = fx.astype(jnp.int32) + _M_RANGE

    b_io = jax.lax.broadcasted_iota(jnp.int32, (1, _K, 1), 1)
    iy3 = iy[:, None, :]
    gy3 = gy[:, None, :]
    ix3 = ix[:, None, :]
    gx3 = gx[:, None, :]
    wy7 = (jnp.where(iy3 == b_io, 1.0 - gy3, 0.0) +
           jnp.where(iy3 + 1 == b_io, gy3, 0.0))
    wx7 = (jnp.where(ix3 == b_io, 1.0 - gx3, 0.0) +
           jnp.where(ix3 + 1 == b_io, gx3, 0.0))
    m_halo = (jnp.repeat(wy7, _K, axis=1) *
              jnp.tile(wx7, (1, _K, 1)))

    for dx in range(_K):
        mxs_ref[dx] = m_halo[:, :, dx:dx + w]

    sl_r = slice(_M_RANGE, _M_RANGE + bh)
    sl_c = slice(_M_RANGE, _M_RANGE + w)
    iyc = iy[sl_r, sl_c]
    gyc = gy[sl_r, sl_c]
    ixc = ix[sl_r, sl_c]
    gxc = gx[sl_r, sl_c]
    wyc = [jnp.where(iyc == j, 1.0 - gyc, 0.0) +
           jnp.where(iyc == j - 1, gyc, 0.0) for j in range(_K)]
    wxc = [jnp.where(ixc == j, 1.0 - gxc, 0.0) +
           jnp.where(ixc == j - 1, gxc, 0.0) for j in range(_K)]
    for n in range(_NC):
        mm_ref[0, n] = wyc[n // _K] * wxc[n % _K]

    kv = [kt3_ref[t][None] for t in range(_NC)]
    om_rows = []
    for y in range(bh):
        om_rows.append(functools.reduce(
            lambda a, b: a + b,
            [kv[dy * _K + dx] * mxs_ref[dx, y + dy]
             for dy in range(_K) for dx in range(_K)]))
    om_cat = jnp.concatenate([r[0] for r in om_rows], axis=1)

    a_flat = jnp.dot(kt_ref[...], om_cat,
                     preferred_element_type=jnp.float32)
    a_all = jnp.stack([a_flat[:, y * w:(y + 1) * w] for y in range(bh)],
                      axis=0)

    imc = im_ref[0, 0]
    for c in range(3):
        terms = [a_all[:, dy * _K + dx, :] * imc[c, dy:dy + bh, dx:dx + w]
                 for dy in range(_K) for dx in range(_K)]
        pred_ref[0, c] = functools.reduce(lambda a, b: a + b, terms)


def kernel(im_input, im_output, gt_motion, m_kernel):
    del im_output
    b, _, h, w = gt_motion.shape
    bh = 32
    nblk = h // bh
    rows = bh + 2 * _M_RANGE
    wp = w + 2 * _M_RANGE

    im = im_input[:, -3:]
    gtp = jnp.pad(gt_motion, ((0, 0), (0, 0),
                              (_M_RANGE, _M_RANGE), (_M_RANGE, _M_RANGE)),
                  constant_values=1.0e4)
    imp = jnp.pad(im, ((0, 0), (0, 0),
                       (_M_RANGE, _M_RANGE), (_M_RANGE, _M_RANGE)))
    row_idx = (jnp.arange(nblk) * bh)[:, None] + jnp.arange(rows)[None, :]
    gt_blk = gtp[:, :, row_idx, :].transpose(0, 2, 1, 3, 4)
    im_blk = imp[:, :, row_idx, :].transpose(0, 2, 1, 3, 4)

    k2 = m_kernel.reshape(_NC, _NC)
    kt = k2.T
    kt3 = kt[:, :, None]

    grid = (b, nblk)
    out_shape = [
        jax.ShapeDtypeStruct((b, _NC, h, w), jnp.float32),
        jax.ShapeDtypeStruct((b, 3, h, w), jnp.float32),
    ]
    mm, pred = pl.pallas_call(
        functools.partial(_body, bh=bh, h=h, w=w),
        grid=grid,
        in_specs=[
            pl.BlockSpec((1, 1, 2, rows, wp), lambda bb, ii: (bb, ii, 0, 0, 0)),
            pl.BlockSpec((1, 1, 3, rows, wp), lambda bb, ii: (bb, ii, 0, 0, 0)),
            pl.BlockSpec((_NC, _NC), lambda bb, ii: (0, 0)),
            pl.BlockSpec((_NC, _NC, 1), lambda bb, ii: (0, 0, 0)),
        ],
        out_specs=[
            pl.BlockSpec((1, _NC, bh, w), lambda bb, ii: (bb, 0, ii, 0)),
            pl.BlockSpec((1, 3, bh, w), lambda bb, ii: (bb, 0, ii, 0)),
        ],
        out_shape=out_shape,
        scratch_shapes=[
            pltpu.VMEM((_K, rows, _NC, w), jnp.float32),
        ],
        compiler_params=pltpu.CompilerParams(
            dimension_semantics=("parallel", "arbitrary"),
            vmem_limit_bytes=56 * 1024 * 1024,
        ),
        name="gtnet_fused",
    )(gt_blk, im_blk, kt, kt3)
    return pred, mm


# speedup vs baseline: 1.1965x; 1.1965x over previous
import functools

import jax
import jax.numpy as jnp
from jax.experimental import pallas as pl
from jax.experimental.pallas import tpu as pltpu

_M_RANGE = 3
_K = 7
_NC = _K * _K


def _body(gt_ref, im_ref, kt_ref, kt3_ref, mm_ref, pred_ref, mxs_ref,
          *, bh, h, w):
    rows = bh + 2 * _M_RANGE

    gt = gt_ref[0, 0]
    mx_ = gt[0]
    my_ = gt[1]
    fy = jnp.floor(my_)
    gy = my_ - fy
    iy = fy.astype(jnp.int32) + _M_RANGE
    fx = jnp.floor(mx_)
    gx = mx_ - fx
    ix = fx.astype(jnp.int32) + _M_RANGE

    n_io = jax.lax.broadcasted_iota(jnp.int32, (1, _NC, 1), 1)
    iyn = n_io // _K
    ixn = n_io % _K
    iy3 = iy[:, None, :]
    gy3 = gy[:, None, :]
    ix3 = ix[:, None, :]
    gx3 = gx[:, None, :]
    wy = (jnp.where(iy3 == iyn, 1.0 - gy3, 0.0) +
          jnp.where(iy3 + 1 == iyn, gy3, 0.0))
    wxv = (jnp.where(ix3 == ixn, 1.0 - gx3, 0.0) +
           jnp.where(ix3 + 1 == ixn, gx3, 0.0))
    m_halo = wy * wxv

    mm_ref[0] = m_halo[_M_RANGE:_M_RANGE + bh]

    for dx in range(_K):
        s = dx - _M_RANGE
        if s < 0:
            mxs_ref[dx, :, :, :-s] = jnp.zeros((rows, _NC, -s), jnp.float32)
            mxs_ref[dx, :, :, -s:] = m_halo[:, :, :w + s]
        elif s == 0:
            mxs_ref[dx] = m_halo
        else:
            mxs_ref[dx, :, :, :w - s] = m_halo[:, :, s:]
            mxs_ref[dx, :, :, w - s:] = jnp.zeros((rows, _NC, s), jnp.float32)

    kv = [kt3_ref[t][None] for t in range(_NC)]
    om_rows = []
    for y in range(bh):
        om_rows.append(functools.reduce(
            lambda a, b: a + b,
            [kv[dy * _K + dx] * mxs_ref[dx, y + dy]
             for dy in range(_K) for dx in range(_K)]))
    om_cat = jnp.concatenate([r[0] for r in om_rows], axis=1)

    a_flat = jnp.dot(kt_ref[...], om_cat,
                     preferred_element_type=jnp.float32)
    a_all = jnp.stack([a_flat[:, y * w:(y + 1) * w] for y in range(bh)],
                      axis=0)

    imc = im_ref[0, 0]
    for c in range(3):
        terms = [a_all[:, dy * _K + dx, :] * imc[c, dy:dy + bh, dx:dx + w]
                 for dy in range(_K) for dx in range(_K)]
        pred_ref[0, c] = functools.reduce(lambda a, b: a + b, terms)


def kernel(im_input, im_output, gt_motion, m_kernel):
    del im_output
    b, _, h, w = gt_motion.shape
    bh = 32
    nblk = h // bh
    rows = bh + 2 * _M_RANGE
    wp = w + 2 * _M_RANGE

    im = im_input[:, -3:]
    gtp = jnp.pad(gt_motion, ((0, 0), (0, 0), (_M_RANGE, _M_RANGE), (0, 0)),
                  constant_values=1.0e4)
    imp = jnp.pad(im, ((0, 0), (0, 0),
                       (_M_RANGE, _M_RANGE), (_M_RANGE, _M_RANGE)))
    row_idx = (jnp.arange(nblk) * bh)[:, None] + jnp.arange(rows)[None, :]
    gt_blk = gtp[:, :, row_idx, :].transpose(0, 2, 1, 3, 4)
    im_blk = imp[:, :, row_idx, :].transpose(0, 2, 1, 3, 4)

    k2 = m_kernel.reshape(_NC, _NC)
    kt = k2.T
    kt3 = kt[:, :, None]

    grid = (b, nblk)
    out_shape = [
        jax.ShapeDtypeStruct((b, h, _NC, w), jnp.float32),
        jax.ShapeDtypeStruct((b, 3, h, w), jnp.float32),
    ]
    mm_t, pred = pl.pallas_call(
        functools.partial(_body, bh=bh, h=h, w=w),
        grid=grid,
        in_specs=[
            pl.BlockSpec((1, 1, 2, rows, w), lambda bb, ii: (bb, ii, 0, 0, 0)),
            pl.BlockSpec((1, 1, 3, rows, wp), lambda bb, ii: (bb, ii, 0, 0, 0)),
            pl.BlockSpec((_NC, _NC), lambda bb, ii: (0, 0)),
            pl.BlockSpec((_NC, _NC, 1), lambda bb, ii: (0, 0, 0)),
        ],
        out_specs=[
            pl.BlockSpec((1, bh, _NC, w), lambda bb, ii: (bb, ii, 0, 0)),
            pl.BlockSpec((1, 3, bh, w), lambda bb, ii: (bb, 0, ii, 0)),
        ],
        out_shape=out_shape,
        scratch_shapes=[
            pltpu.VMEM((_K, rows, _NC, w), jnp.float32),
        ],
        compiler_params=pltpu.CompilerParams(
            dimension_semantics=("parallel", "arbitrary"),
            vmem_limit_bytes=56 * 1024 * 1024,
        ),
        name="gtnet_fused",
    )(gt_blk, im_blk, kt, kt3)
    m_mask = mm_t.transpose(0, 2, 1, 3)
    return pred, m_mask


# speedup vs baseline: 1.6083x; 1.3442x over previous
import functools

import jax
import jax.numpy as jnp
from jax.experimental import pallas as pl
from jax.experimental.pallas import tpu as pltpu

_M_RANGE = 3
_K = 7
_NC = _K * _K


def _body(gt_ref, im_ref, kt_ref, kt3_ref, mm_ref, pred_ref, mxs_ref,
          *, bh, h, w):
    rows = bh + 2 * _M_RANGE

    gt = gt_ref[0, 0]
    mx_ = gt[0]
    my_ = gt[1]
    fy = jnp.floor(my_)
    gy = my_ - fy
    iy = fy.astype(jnp.int32) + _M_RANGE
    fx = jnp.floor(mx_)
    gx = mx_ - fx
    ix = fx.astype(jnp.int32) + _M_RANGE

    n_io = jax.lax.broadcasted_iota(jnp.int32, (1, _NC, 1), 1)
    iyn = n_io // _K
    ixn = n_io % _K
    iy3 = iy[:, None, :]
    gy3 = gy[:, None, :]
    ix3 = ix[:, None, :]
    gx3 = gx[:, None, :]
    wy = (jnp.where(iy3 == iyn, 1.0 - gy3, 0.0) +
          jnp.where(iy3 + 1 == iyn, gy3, 0.0))
    wxv = (jnp.where(ix3 == ixn, 1.0 - gx3, 0.0) +
           jnp.where(ix3 + 1 == ixn, gx3, 0.0))
    m_halo = wy * wxv

    mm_ref[0] = m_halo[_M_RANGE:_M_RANGE + bh]

    m_halo16 = m_halo.astype(jnp.bfloat16)
    for dx in range(_K):
        s = dx - _M_RANGE
        if s < 0:
            mxs_ref[dx, :, :, :-s] = jnp.zeros((rows, _NC, -s), jnp.bfloat16)
            mxs_ref[dx, :, :, -s:] = m_halo16[:, :, :w + s]
        elif s == 0:
            mxs_ref[dx] = m_halo16
        else:
            mxs_ref[dx, :, :, :w - s] = m_halo16[:, :, s:]
            mxs_ref[dx, :, :, w - s:] = jnp.zeros((rows, _NC, s), jnp.bfloat16)

    kv = [kt3_ref[t][None] for t in range(_NC)]
    om_rows = []
    for y in range(bh):
        om_rows.append(functools.reduce(
            lambda a, b: a + b,
            [kv[dy * _K + dx] * mxs_ref[dx, y + dy]
             for dy in range(_K) for dx in range(_K)]))
    om_cat = jnp.concatenate([r[0] for r in om_rows], axis=1)

    a_flat = jnp.dot(kt_ref[...].astype(jnp.bfloat16), om_cat,
                     preferred_element_type=jnp.float32)
    a_all = jnp.stack([a_flat[:, y * w:(y + 1) * w] for y in range(bh)],
                      axis=0)

    imc = im_ref[0, 0]
    for c in range(3):
        terms = [a_all[:, dy * _K + dx, :] * imc[c, dy:dy + bh, dx:dx + w]
                 for dy in range(_K) for dx in range(_K)]
        pred_ref[0, c] = functools.reduce(lambda a, b: a + b, terms)


def kernel(im_input, im_output, gt_motion, m_kernel):
    del im_output
    b, _, h, w = gt_motion.shape
    bh = 32
    nblk = h // bh
    rows = bh + 2 * _M_RANGE
    wp = w + 2 * _M_RANGE

    im = im_input[:, -3:]
    gtp = jnp.pad(gt_motion, ((0, 0), (0, 0), (_M_RANGE, _M_RANGE), (0, 0)),
                  constant_values=1.0e4)
    imp = jnp.pad(im, ((0, 0), (0, 0),
                       (_M_RANGE, _M_RANGE), (_M_RANGE, _M_RANGE)))
    row_idx = (jnp.arange(nblk) * bh)[:, None] + jnp.arange(rows)[None, :]
    gt_blk = gtp[:, :, row_idx, :].transpose(0, 2, 1, 3, 4)
    im_blk = imp[:, :, row_idx, :].transpose(0, 2, 1, 3, 4)

    k2 = m_kernel.reshape(_NC, _NC)
    kt = k2.T
    kt3 = kt[:, :, None].astype(jnp.bfloat16)

    grid = (b, nblk)
    out_shape = [
        jax.ShapeDtypeStruct((b, h, _NC, w), jnp.float32),
        jax.ShapeDtypeStruct((b, 3, h, w), jnp.float32),
    ]
    mm_t, pred = pl.pallas_call(
        functools.partial(_body, bh=bh, h=h, w=w),
        grid=grid,
        in_specs=[
            pl.BlockSpec((1, 1, 2, rows, w), lambda bb, ii: (bb, ii, 0, 0, 0)),
            pl.BlockSpec((1, 1, 3, rows, wp), lambda bb, ii: (bb, ii, 0, 0, 0)),
            pl.BlockSpec((_NC, _NC), lambda bb, ii: (0, 0)),
            pl.BlockSpec((_NC, _NC, 1), lambda bb, ii: (0, 0, 0)),
        ],
        out_specs=[
            pl.BlockSpec((1, bh, _NC, w), lambda bb, ii: (bb, ii, 0, 0)),
            pl.BlockSpec((1, 3, bh, w), lambda bb, ii: (bb, 0, ii, 0)),
        ],
        out_shape=out_shape,
        scratch_shapes=[
            pltpu.VMEM((_K, rows, _NC, w), jnp.bfloat16),
        ],
        compiler_params=pltpu.CompilerParams(
            dimension_semantics=("parallel", "arbitrary"),
            vmem_limit_bytes=56 * 1024 * 1024,
        ),
        name="gtnet_fused",
    )(gt_blk, im_blk, kt, kt3)
    m_mask = mm_t.transpose(0, 2, 1, 3)
    return pred, m_mask
